# Initial kernel scaffold; baseline (speedup 1.0000x reference)
import jax
import jax.numpy as jnp
from jax.experimental import pallas as pl


def kernel(x, edge_index, W1, a1s, a1d, b1, W2, a2s, a2d, b2, W3, a3s, a3d, b3):
    raise NotImplementedError("write your pallas kernel here")


# speedup vs baseline: 15.2828x; 15.2828x over previous
import functools

import jax
import jax.numpy as jnp
from jax import lax
from jax.experimental import pallas as pl
from jax.experimental.pallas import tpu as pltpu
from jax.experimental.pallas import tpu_sc as plsc

N = 10000
NP = 10240
E2 = 330000
EBATCH = 64
E2P = 331776
D_IN, H1, H2, NCLS = 128, 256, 128, 7
HP = 144
HP3 = 16
BLK = 512
NBLK = NP // BLK
RPT = NP // 16


def _sc_edge_sweep(hp, nb_core, nbt, src_off):
  g = hp // 16
  mesh = plsc.VectorSubcoreMesh(core_axis_name="c", subcore_axis_name="s")

  @functools.partial(
      pl.kernel,
      out_type=jax.ShapeDtypeStruct((2 * NP, hp), jnp.float32),
      mesh=mesh,
      compiler_params=pltpu.CompilerParams(needs_layout_passes=False,
                                           use_tc_tiling_on_sc=False),
      scratch_types=[
          pltpu.VMEM((NP,), jnp.float32),
          pltpu.VMEM((NP,), jnp.float32),
          pltpu.VMEM((EBATCH,), jnp.int32),
          pltpu.VMEM((EBATCH,), jnp.int32),
          pltpu.VMEM((EBATCH, hp), jnp.float32),
          pltpu.VMEM((EBATCH,), jnp.float32),
          pltpu.VMEM_SHARED((NP, hp), jnp.float32),
          pltpu.SemaphoreType.DMA,
      ],
  )
  def sweep(table, s2, d2, srcs, dsts, zeros, out,
            s_v, d_v, src_v, dst_v, rows_v, p_v, accum, sem):
    c = lax.axis_index("c")
    t = lax.axis_index("s")
    off = c * src_off
    pltpu.sync_copy(s2, s_v)
    pltpu.sync_copy(d2, d_v)
    pltpu.sync_copy(zeros.at[pl.ds(t * RPT, RPT)],
                    accum.at[pl.ds(t * RPT, RPT)])
    plsc.subcore_barrier()

    def batch(j, carry):
      row = c * nb_core + j
      pltpu.sync_copy(srcs.at[row], src_v)
      pltpu.sync_copy(dsts.at[row], dst_v)
      gat = pltpu.async_copy(table.at[src_v], rows_v, sem)
      for k in range(EBATCH // 16):
        sl = pl.ds(k * 16, 16)
        e = (plsc.load_gather(s_v, [src_v[sl] - off])
             + plsc.load_gather(d_v, [dst_v[sl]]))
        e = jnp.where(e > 0, e, e * 0.2)
        p_v[sl] = jnp.exp(e)
      gat.wait()

      def scale(i, carry2):
        pb = plsc.load_gather(p_v, [jnp.full((16,), i, jnp.int32)])
        for q in range(g):
          sl2 = pl.ds(q * 16, 16)
          rows_v[i, sl2] = rows_v[i, sl2] * pb
        return carry2

      lax.fori_loop(0, EBATCH, scale, 0)
      pltpu.sync_copy(rows_v, accum.at[dst_v], add=True)
      return carry

    lax.fori_loop(t * nbt, (t + 1) * nbt, batch, 0)
    plsc.subcore_barrier()
    pltpu.sync_copy(accum.at[pl.ds(t * RPT, RPT)],
                    out.at[pl.ds(c * NP + t * RPT, RPT)])

  return sweep


_SC1 = _sc_edge_sweep(HP, E2P // EBATCH, E2P // EBATCH // 16, NP)
_SC2 = _sc_edge_sweep(HP, E2P // EBATCH // 2, E2P // EBATCH // 32, 0)
_SC3 = _sc_edge_sweep(HP3, E2P // EBATCH // 2, E2P // EBATCH // 32, 0)


def _valid_col(i):
  rows = i * BLK + lax.broadcasted_iota(jnp.int32, (BLK, 1), 0)
  return (rows < N).astype(jnp.float32)


def _sd_block(h, a_ref):
  s_col = jnp.sum(h * a_ref[0:1, :], axis=1, keepdims=True)
  d_col = jnp.sum(h * a_ref[1:2, :], axis=1, keepdims=True)
  col = lax.broadcasted_iota(jnp.int32, (BLK, 128), 1)
  return jnp.where(col == 0, s_col, jnp.where(col == 1, d_col, 0.0))


def _tc_layer1(xp, w1, a1):
  def body(x_ref, w_ref, a_ref, t_ref, sd_ref):
    i = pl.program_id(0)
    blk = i % NBLK
    h = jnp.dot(x_ref[...], w_ref[...], preferred_element_type=jnp.float32)
    valid = _valid_col(blk)
    sd_ref[...] = _sd_block(h, a_ref)
    hsel = jnp.where(i < NBLK, h[:, :128], h[:, 128:])
    t_ref[...] = jnp.concatenate(
        [hsel, valid, jnp.zeros((BLK, HP - 129), jnp.float32)], axis=1)

  return pl.pallas_call(
      body,
      grid=(2 * NBLK,),
      in_specs=[
          pl.BlockSpec((BLK, D_IN), lambda i: (i % NBLK, 0)),
          pl.BlockSpec((D_IN, H1), lambda i: (0, 0)),
          pl.BlockSpec((2, H1), lambda i: (0, 0)),
      ],
      out_specs=[
          pl.BlockSpec((BLK, HP), lambda i: (i, 0)),
          pl.BlockSpec((BLK, 128), lambda i: (i % NBLK, 0)),
      ],
      out_shape=[
          jax.ShapeDtypeStruct((2 * NP, HP), jnp.float32),
          jax.ShapeDtypeStruct((NP, 128), jnp.float32),
      ],
  )(xp, w1, a1)


def _elu(v):
  return jnp.where(v > 0, v, jnp.exp(v) - 1.0)


def _tc_layer2(u1, w2, a2, b1r):
  def body(u_ref, w_ref, a_ref, b_ref, t_ref, sd_ref):
    i = pl.program_id(0)
    ua = u_ref[0]
    ub = u_ref[1]
    ha = ua[:, :128] / ua[:, 128:129]
    hb = ub[:, :128] / ub[:, 128:129]
    valid = _valid_col(i)
    x2 = jnp.concatenate([ha, hb], axis=1) + b_ref[...]
    x2 = jnp.where(valid > 0, _elu(x2), 0.0)
    h = jnp.dot(x2, w_ref[...], preferred_element_type=jnp.float32)
    sd_ref[...] = _sd_block(h, a_ref)
    t_ref[...] = jnp.concatenate(
        [h, valid, jnp.zeros((BLK, HP - 129), jnp.float32)], axis=1)

  return pl.pallas_call(
      body,
      grid=(NBLK,),
      in_specs=[
          pl.BlockSpec((2, BLK, HP), lambda i: (0, i, 0)),
          pl.BlockSpec((H1, H2), lambda i: (0, 0)),
          pl.BlockSpec((2, H2), lambda i: (0, 0)),
          pl.BlockSpec((1, H1), lambda i: (0, 0)),
      ],
      out_specs=[
          pl.BlockSpec((BLK, HP), lambda i: (i, 0)),
          pl.BlockSpec((BLK, 128), lambda i: (i, 0)),
      ],
      out_shape=[
          jax.ShapeDtypeStruct((NP, HP), jnp.float32),
          jax.ShapeDtypeStruct((NP, 128), jnp.float32),
      ],
  )(u1, w2, a2, b1r)


def _tc_layer3(u2, w3p, a3p, b2r):
  def body(u_ref, w_ref, a_ref, b_ref, t_ref, sd_ref):
    i = pl.program_id(0)
    acc = u_ref[0] + u_ref[1]
    valid = _valid_col(i)
    x3 = acc[:, :128] / acc[:, 128:129] + b_ref[...]
    x3 = jnp.where(valid > 0, _elu(x3), 0.0)
    h = jnp.dot(x3, w_ref[...], preferred_element_type=jnp.float32)
    s_col = jnp.sum(h * a_ref[0:1, :], axis=1, keepdims=True)
    d_col = jnp.sum(h * a_ref[1:2, :], axis=1, keepdims=True)
    col = lax.broadcasted_iota(jnp.int32, (BLK, 128), 1)
    sd_ref[...] = jnp.where(col == 0, s_col, jnp.where(col == 1, d_col, 0.0))
    col16 = lax.broadcasted_iota(jnp.int32, (BLK, HP3), 1)
    t_ref[...] = h + jnp.where(col16 == NCLS, valid, 0.0)

  return pl.pallas_call(
      body,
      grid=(NBLK,),
      in_specs=[
          pl.BlockSpec((2, BLK, HP), lambda i: (0, i, 0)),
          pl.BlockSpec((H2, HP3), lambda i: (0, 0)),
          pl.BlockSpec((2, HP3), lambda i: (0, 0)),
          pl.BlockSpec((1, H2), lambda i: (0, 0)),
      ],
      out_specs=[
          pl.BlockSpec((BLK, HP3), lambda i: (i, 0)),
          pl.BlockSpec((BLK, 128), lambda i: (i, 0)),
      ],
      out_shape=[
          jax.ShapeDtypeStruct((NP, HP3), jnp.float32),
          jax.ShapeDtypeStruct((NP, 128), jnp.float32),
      ],
  )(u2, w3p, a3p, b2r)


def _tc_final(u3, b3p):
  def body(u_ref, b_ref, o_ref):
    i = pl.program_id(0)
    acc = u_ref[0] + u_ref[1]
    valid = _valid_col(i)
    o = acc / acc[:, NCLS:NCLS + 1] + b_ref[...]
    o_ref[...] = jnp.where(valid > 0, o, 0.0)

  return pl.pallas_call(
      body,
      grid=(NBLK,),
      in_specs=[
          pl.BlockSpec((2, BLK, HP3), lambda i: (0, i, 0)),
          pl.BlockSpec((1, HP3), lambda i: (0, 0)),
      ],
      out_specs=pl.BlockSpec((BLK, HP3), lambda i: (i, 0)),
      out_shape=jax.ShapeDtypeStruct((NP, HP3), jnp.float32),
  )(u3, b3p)


def kernel(x, edge_index, W1, a1s, a1d, b1, W2, a2s, a2d, b2, W3, a3s, a3d,
           b3):
  loop = jnp.arange(N, dtype=jnp.int32)
  src = jnp.concatenate([edge_index[0].astype(jnp.int32), loop])
  dst = jnp.concatenate([edge_index[1].astype(jnp.int32), loop])
  pad = E2P - E2
  srcp = jnp.concatenate([src, jnp.full((pad,), N, jnp.int32)])
  dstp = jnp.concatenate([dst, jnp.full((pad,), N, jnp.int32)])
  src1 = jnp.concatenate([srcp, srcp + NP]).reshape(-1, EBATCH)
  dst1 = jnp.concatenate([dstp, dstp]).reshape(-1, EBATCH)
  src23 = srcp.reshape(-1, EBATCH)
  dst23 = dstp.reshape(-1, EBATCH)
  z144 = jnp.zeros((NP, HP), jnp.float32)
  z16 = jnp.zeros((NP, HP3), jnp.float32)

  xp = jnp.pad(x, ((0, NP - N), (0, 0)))
  a1 = jnp.stack([a1s, a1d])
  a2 = jnp.stack([a2s, a2d])
  w3p = jnp.pad(W3, ((0, 0), (0, HP3 - NCLS)))
  a3p = jnp.stack([jnp.pad(a3s, (0, HP3 - NCLS)),
                   jnp.pad(a3d, (0, HP3 - NCLS))])
  b1r = b1.reshape(1, H1)
  b2r = b2.reshape(1, H2)
  b3p = jnp.pad(b3, (0, HP3 - NCLS)).reshape(1, HP3)

  t1, sd1 = _tc_layer1(xp, W1, a1)
  u1 = _SC1(t1, sd1[:, 0], sd1[:, 1], src1, dst1, z144)

  t2, sd2 = _tc_layer2(u1.reshape(2, NP, HP), W2, a2, b1r)
  u2 = _SC2(t2, sd2[:, 0], sd2[:, 1], src23, dst23, z144)

  t3, sd3 = _tc_layer3(u2.reshape(2, NP, HP), w3p, a3p, b2r)
  u3 = _SC3(t3, sd3[:, 0], sd3[:, 1], src23, dst23, z16)

  out = _tc_final(u3.reshape(2, NP, HP3), b3p)
  return out[:N, :NCLS]


# speedup vs baseline: 24.6485x; 1.6128x over previous
import functools

import jax
import jax.numpy as jnp
from jax import lax
from jax.experimental import pallas as pl
from jax.experimental.pallas import tpu as pltpu
from jax.experimental.pallas import tpu_sc as plsc

N = 10000
NP = 10240
E2 = 330000
EBATCH = 48
CHUNKB = 8
E2P = 331776
D_IN, H1, H2, NCLS = 128, 256, 128, 7
HP = 144
HP3 = 16
BLK = 512
NBLK = NP // BLK
RPT = NP // 16


def _sc_edge_sweep(hp, nb_core, nbt, src_off):
  g = hp // 16
  nchunk = nbt // CHUNKB
  mesh = plsc.VectorSubcoreMesh(core_axis_name="c", subcore_axis_name="s")

  @functools.partial(
      pl.kernel,
      out_type=jax.ShapeDtypeStruct((2 * NP, hp), jnp.float32),
      mesh=mesh,
      compiler_params=pltpu.CompilerParams(needs_layout_passes=False,
                                           use_tc_tiling_on_sc=False),
      scratch_types=[
          pltpu.VMEM((NP,), jnp.float32),
          pltpu.VMEM((NP,), jnp.float32),
          pltpu.VMEM((CHUNKB, EBATCH), jnp.int32),
          pltpu.VMEM((CHUNKB, EBATCH), jnp.int32),
          [pltpu.VMEM((EBATCH, hp), jnp.float32)] * 2,
          [pltpu.VMEM((EBATCH,), jnp.float32)] * 2,
          pltpu.VMEM_SHARED((NP, hp), jnp.float32),
          [pltpu.SemaphoreType.DMA] * 2,
          [pltpu.SemaphoreType.DMA] * 2,
      ],
  )
  def sweep(table, s2, d2, srcs, dsts, zeros, out,
            s_v, d_v, src_ch, dst_ch, rows, ps, accum, gsem, ssem):
    c = lax.axis_index("c")
    t = lax.axis_index("s")
    off = c * src_off
    pltpu.sync_copy(s2, s_v)
    pltpu.sync_copy(d2, d_v)
    pltpu.sync_copy(zeros.at[pl.ds(t * RPT, RPT)],
                    accum.at[pl.ds(t * RPT, RPT)])
    plsc.subcore_barrier()

    def chunk(q, carry):
      row = c * nb_core + q * CHUNKB
      pltpu.sync_copy(srcs.at[pl.ds(row, CHUNKB)], src_ch)
      pltpu.sync_copy(dsts.at[pl.ds(row, CHUNKB)], dst_ch)
      gats = [None, None]
      scats = [None, None]
      gats[0] = pltpu.async_copy(table.at[src_ch.at[0]], rows[0], gsem[0])
      for b in range(CHUNKB):
        par = b % 2
        nxt = (b + 1) % 2
        if b + 1 < CHUNKB:
          if scats[nxt] is not None:
            scats[nxt].wait()
          gats[nxt] = pltpu.async_copy(table.at[src_ch.at[b + 1]],
                                       rows[nxt], gsem[nxt])
        for k in range(EBATCH // 16):
          sl = pl.ds(k * 16, 16)
          e = (plsc.load_gather(s_v, [src_ch[b, sl] - off])
               + plsc.load_gather(d_v, [dst_ch[b, sl]]))
          e = jnp.where(e > 0, e, e * 0.2)
          ps[par][sl] = jnp.exp(e)
        gats[par].wait()

        def scale(i, carry2, par=par):
          pb = plsc.load_gather(ps[par], [jnp.full((16,), i, jnp.int32)])
          for v in range(g):
            sl2 = pl.ds(v * 16, 16)
            rows[par][i, sl2] = rows[par][i, sl2] * pb
          return carry2

        lax.fori_loop(0, EBATCH, scale, 0)
        scats[par] = pltpu.async_copy(rows[par], accum.at[dst_ch.at[b]],
                                      ssem[par], add=True)
      scats[0].wait()
      scats[1].wait()
      return carry

    lax.fori_loop(t * nchunk, (t + 1) * nchunk, chunk, 0)
    plsc.subcore_barrier()
    pltpu.sync_copy(accum.at[pl.ds(t * RPT, RPT)],
                    out.at[pl.ds(c * NP + t * RPT, RPT)])

  return sweep


_SC1 = _sc_edge_sweep(HP, E2P // EBATCH, E2P // EBATCH // 16, NP)
_SC2 = _sc_edge_sweep(HP, E2P // EBATCH // 2, E2P // EBATCH // 32, 0)
_SC3 = _sc_edge_sweep(HP3, E2P // EBATCH // 2, E2P // EBATCH // 32, 0)


def _valid_col(i):
  rows = i * BLK + lax.broadcasted_iota(jnp.int32, (BLK, 1), 0)
  return (rows < N).astype(jnp.float32)


def _sd_block(h, a_ref):
  s_col = jnp.sum(h * a_ref[0:1, :], axis=1, keepdims=True)
  d_col = jnp.sum(h * a_ref[1:2, :], axis=1, keepdims=True)
  col = lax.broadcasted_iota(jnp.int32, (BLK, 128), 1)
  return jnp.where(col == 0, s_col, jnp.where(col == 1, d_col, 0.0))


def _tc_layer1(xp, w1, a1):
  def body(x_ref, w_ref, a_ref, t_ref, sd_ref):
    i = pl.program_id(0)
    blk = i % NBLK
    h = jnp.dot(x_ref[...], w_ref[...], preferred_element_type=jnp.float32)
    valid = _valid_col(blk)
    sd_ref[...] = _sd_block(h, a_ref)
    hsel = jnp.where(i < NBLK, h[:, :128], h[:, 128:])
    t_ref[...] = jnp.concatenate(
        [hsel, valid, jnp.zeros((BLK, HP - 129), jnp.float32)], axis=1)

  return pl.pallas_call(
      body,
      grid=(2 * NBLK,),
      in_specs=[
          pl.BlockSpec((BLK, D_IN), lambda i: (i % NBLK, 0)),
          pl.BlockSpec((D_IN, H1), lambda i: (0, 0)),
          pl.BlockSpec((2, H1), lambda i: (0, 0)),
      ],
      out_specs=[
          pl.BlockSpec((BLK, HP), lambda i: (i, 0)),
          pl.BlockSpec((BLK, 128), lambda i: (i % NBLK, 0)),
      ],
      out_shape=[
          jax.ShapeDtypeStruct((2 * NP, HP), jnp.float32),
          jax.ShapeDtypeStruct((NP, 128), jnp.float32),
      ],
  )(xp, w1, a1)


def _elu(v):
  return jnp.where(v > 0, v, jnp.exp(v) - 1.0)


def _tc_layer2(u1, w2, a2, b1r):
  def body(u_ref, w_ref, a_ref, b_ref, t_ref, sd_ref):
    i = pl.program_id(0)
    ua = u_ref[0]
    ub = u_ref[1]
    ha = ua[:, :128] / ua[:, 128:129]
    hb = ub[:, :128] / ub[:, 128:129]
    valid = _valid_col(i)
    x2 = jnp.concatenate([ha, hb], axis=1) + b_ref[...]
    x2 = jnp.where(valid > 0, _elu(x2), 0.0)
    h = jnp.dot(x2, w_ref[...], preferred_element_type=jnp.float32)
    sd_ref[...] = _sd_block(h, a_ref)
    t_ref[...] = jnp.concatenate(
        [h, valid, jnp.zeros((BLK, HP - 129), jnp.float32)], axis=1)

  return pl.pallas_call(
      body,
      grid=(NBLK,),
      in_specs=[
          pl.BlockSpec((2, BLK, HP), lambda i: (0, i, 0)),
          pl.BlockSpec((H1, H2), lambda i: (0, 0)),
          pl.BlockSpec((2, H2), lambda i: (0, 0)),
          pl.BlockSpec((1, H1), lambda i: (0, 0)),
      ],
      out_specs=[
          pl.BlockSpec((BLK, HP), lambda i: (i, 0)),
          pl.BlockSpec((BLK, 128), lambda i: (i, 0)),
      ],
      out_shape=[
          jax.ShapeDtypeStruct((NP, HP), jnp.float32),
          jax.ShapeDtypeStruct((NP, 128), jnp.float32),
      ],
  )(u1, w2, a2, b1r)


def _tc_layer3(u2, w3p, a3p, b2r):
  def body(u_ref, w_ref, a_ref, b_ref, t_ref, sd_ref):
    i = pl.program_id(0)
    acc = u_ref[0] + u_ref[1]
    valid = _valid_col(i)
    x3 = acc[:, :128] / acc[:, 128:129] + b_ref[...]
    x3 = jnp.where(valid > 0, _elu(x3), 0.0)
    h = jnp.dot(x3, w_ref[...], preferred_element_type=jnp.float32)
    s_col = jnp.sum(h * a_ref[0:1, :], axis=1, keepdims=True)
    d_col = jnp.sum(h * a_ref[1:2, :], axis=1, keepdims=True)
    col = lax.broadcasted_iota(jnp.int32, (BLK, 128), 1)
    sd_ref[...] = jnp.where(col == 0, s_col, jnp.where(col == 1, d_col, 0.0))
    col16 = lax.broadcasted_iota(jnp.int32, (BLK, HP3), 1)
    t_ref[...] = h + jnp.where(col16 == NCLS, valid, 0.0)

  return pl.pallas_call(
      body,
      grid=(NBLK,),
      in_specs=[
          pl.BlockSpec((2, BLK, HP), lambda i: (0, i, 0)),
          pl.BlockSpec((H2, HP3), lambda i: (0, 0)),
          pl.BlockSpec((2, HP3), lambda i: (0, 0)),
          pl.BlockSpec((1, H2), lambda i: (0, 0)),
      ],
      out_specs=[
          pl.BlockSpec((BLK, HP3), lambda i: (i, 0)),
          pl.BlockSpec((BLK, 128), lambda i: (i, 0)),
      ],
      out_shape=[
          jax.ShapeDtypeStruct((NP, HP3), jnp.float32),
          jax.ShapeDtypeStruct((NP, 128), jnp.float32),
      ],
  )(u2, w3p, a3p, b2r)


def _tc_final(u3, b3p):
  def body(u_ref, b_ref, o_ref):
    i = pl.program_id(0)
    acc = u_ref[0] + u_ref[1]
    valid = _valid_col(i)
    o = acc / acc[:, NCLS:NCLS + 1] + b_ref[...]
    o_ref[...] = jnp.where(valid > 0, o, 0.0)

  return pl.pallas_call(
      body,
      grid=(NBLK,),
      in_specs=[
          pl.BlockSpec((2, BLK, HP3), lambda i: (0, i, 0)),
          pl.BlockSpec((1, HP3), lambda i: (0, 0)),
      ],
      out_specs=pl.BlockSpec((BLK, HP3), lambda i: (i, 0)),
      out_shape=jax.ShapeDtypeStruct((NP, HP3), jnp.float32),
  )(u3, b3p)


def kernel(x, edge_index, W1, a1s, a1d, b1, W2, a2s, a2d, b2, W3, a3s, a3d,
           b3):
  loop = jnp.arange(N, dtype=jnp.int32)
  src = jnp.concatenate([edge_index[0].astype(jnp.int32), loop])
  dst = jnp.concatenate([edge_index[1].astype(jnp.int32), loop])
  pad = E2P - E2
  srcp = jnp.concatenate([src, jnp.full((pad,), N, jnp.int32)])
  dstp = jnp.concatenate([dst, jnp.full((pad,), N, jnp.int32)])
  src1 = jnp.concatenate([srcp, srcp + NP]).reshape(-1, EBATCH)
  dst1 = jnp.concatenate([dstp, dstp]).reshape(-1, EBATCH)
  src23 = srcp.reshape(-1, EBATCH)
  dst23 = dstp.reshape(-1, EBATCH)
  z144 = jnp.zeros((NP, HP), jnp.float32)
  z16 = jnp.zeros((NP, HP3), jnp.float32)

  xp = jnp.pad(x, ((0, NP - N), (0, 0)))
  a1 = jnp.stack([a1s, a1d])
  a2 = jnp.stack([a2s, a2d])
  w3p = jnp.pad(W3, ((0, 0), (0, HP3 - NCLS)))
  a3p = jnp.stack([jnp.pad(a3s, (0, HP3 - NCLS)),
                   jnp.pad(a3d, (0, HP3 - NCLS))])
  b1r = b1.reshape(1, H1)
  b2r = b2.reshape(1, H2)
  b3p = jnp.pad(b3, (0, HP3 - NCLS)).reshape(1, HP3)

  t1, sd1 = _tc_layer1(xp, W1, a1)
  u1 = _SC1(t1, sd1[:, 0], sd1[:, 1], src1, dst1, z144)

  t2, sd2 = _tc_layer2(u1.reshape(2, NP, HP), W2, a2, b1r)
  u2 = _SC2(t2, sd2[:, 0], sd2[:, 1], src23, dst23, z144)

  t3, sd3 = _tc_layer3(u2.reshape(2, NP, HP), w3p, a3p, b2r)
  u3 = _SC3(t3, sd3[:, 0], sd3[:, 1], src23, dst23, z16)

  out = _tc_final(u3.reshape(2, NP, HP3), b3p)
  return out[:N, :NCLS]


# speedup vs baseline: 25.3406x; 1.0281x over previous
import functools

import jax
import jax.numpy as jnp
from jax import lax
from jax.experimental import pallas as pl
from jax.experimental.pallas import tpu as pltpu
from jax.experimental.pallas import tpu_sc as plsc

N = 10000
NP = 10240
E2 = 330000
EBATCH = 48
EBATCH3 = 128
E2P = 331776
D_IN, H1, H2, NCLS = 128, 256, 128, 7
HP = 144
HP3 = 16
BLK = 512
NBLK = NP // BLK
RPT = NP // 16


def _sc_edge_sweep(hp, nb_core, nbt, src_off, eb, cb):
  g = hp // 16
  nchunk = nbt // cb
  mesh = plsc.VectorSubcoreMesh(core_axis_name="c", subcore_axis_name="s")

  @functools.partial(
      pl.kernel,
      out_type=jax.ShapeDtypeStruct((2 * NP, hp), jnp.float32),
      mesh=mesh,
      compiler_params=pltpu.CompilerParams(needs_layout_passes=False,
                                           use_tc_tiling_on_sc=False),
      scratch_types=[
          pltpu.VMEM((NP,), jnp.float32),
          pltpu.VMEM((NP,), jnp.float32),
          pltpu.VMEM((cb, eb), jnp.int32),
          pltpu.VMEM((cb, eb), jnp.int32),
          [pltpu.VMEM((eb, hp), jnp.float32)] * 2,
          [pltpu.VMEM((eb,), jnp.float32)] * 2,
          pltpu.VMEM_SHARED((NP, hp), jnp.float32),
          [pltpu.SemaphoreType.DMA] * 2,
          [pltpu.SemaphoreType.DMA] * 2,
      ],
  )
  def sweep(table, s2, d2, srcs, dsts, zeros, out,
            s_v, d_v, src_ch, dst_ch, rows, ps, accum, gsem, ssem):
    c = lax.axis_index("c")
    t = lax.axis_index("s")
    off = c * src_off
    pltpu.sync_copy(s2, s_v)
    pltpu.sync_copy(d2, d_v)
    pltpu.sync_copy(zeros.at[pl.ds(t * RPT, RPT)],
                    accum.at[pl.ds(t * RPT, RPT)])
    plsc.subcore_barrier()

    def chunk(q, carry):
      row = c * nb_core + q * cb
      pltpu.sync_copy(srcs.at[pl.ds(row, cb)], src_ch)
      pltpu.sync_copy(dsts.at[pl.ds(row, cb)], dst_ch)
      gats = [None, None]
      scats = [None, None]
      gats[0] = pltpu.async_copy(table.at[src_ch.at[0]], rows[0], gsem[0])
      for b in range(cb):
        par = b % 2
        nxt = (b + 1) % 2
        if b + 1 < cb:
          if scats[nxt] is not None:
            scats[nxt].wait()
          gats[nxt] = pltpu.async_copy(table.at[src_ch.at[b + 1]],
                                       rows[nxt], gsem[nxt])
        for k in range(eb // 16):
          sl = pl.ds(k * 16, 16)
          e = (plsc.load_gather(s_v, [src_ch[b, sl] - off])
               + plsc.load_gather(d_v, [dst_ch[b, sl]]))
          e = jnp.where(e > 0, e, e * 0.2)
          ps[par][sl] = jnp.exp(e)
        gats[par].wait()

        def scale(i, carry2, par=par):
          pb = plsc.load_gather(ps[par], [jnp.full((16,), i, jnp.int32)])
          for v in range(g):
            sl2 = pl.ds(v * 16, 16)
            rows[par][i, sl2] = rows[par][i, sl2] * pb
          return carry2

        lax.fori_loop(0, eb, scale, 0, unroll=2)
        scats[par] = pltpu.async_copy(rows[par], accum.at[dst_ch.at[b]],
                                      ssem[par], add=True)
      scats[0].wait()
      scats[1].wait()
      return carry

    lax.fori_loop(t * nchunk, (t + 1) * nchunk, chunk, 0)
    plsc.subcore_barrier()
    pltpu.sync_copy(accum.at[pl.ds(t * RPT, RPT)],
                    out.at[pl.ds(c * NP + t * RPT, RPT)])

  return sweep


_SC1 = _sc_edge_sweep(HP, E2P // EBATCH, E2P // EBATCH // 16, NP, EBATCH, 8)
_SC2 = _sc_edge_sweep(HP, E2P // EBATCH // 2, E2P // EBATCH // 32, 0, EBATCH, 8)
_SC3 = _sc_edge_sweep(HP3, E2P // EBATCH3 // 2, E2P // EBATCH3 // 32, 0,
                      EBATCH3, 3)


def _valid_col(i):
  rows = i * BLK + lax.broadcasted_iota(jnp.int32, (BLK, 1), 0)
  return (rows < N).astype(jnp.float32)


def _sd_block(h, a_ref):
  s_col = jnp.sum(h * a_ref[0:1, :], axis=1, keepdims=True)
  d_col = jnp.sum(h * a_ref[1:2, :], axis=1, keepdims=True)
  col = lax.broadcasted_iota(jnp.int32, (BLK, 128), 1)
  return jnp.where(col == 0, s_col, jnp.where(col == 1, d_col, 0.0))


def _tc_layer1(xp, w1, a1):
  def body(x_ref, w_ref, a_ref, t_ref, sd_ref):
    i = pl.program_id(0)
    blk = i % NBLK
    h = jnp.dot(x_ref[...], w_ref[...], preferred_element_type=jnp.float32)
    valid = _valid_col(blk)
    sd_ref[...] = _sd_block(h, a_ref)
    hsel = jnp.where(i < NBLK, h[:, :128], h[:, 128:])
    t_ref[...] = jnp.concatenate(
        [hsel, valid, jnp.zeros((BLK, HP - 129), jnp.float32)], axis=1)

  return pl.pallas_call(
      body,
      grid=(2 * NBLK,),
      in_specs=[
          pl.BlockSpec((BLK, D_IN), lambda i: (i % NBLK, 0)),
          pl.BlockSpec((D_IN, H1), lambda i: (0, 0)),
          pl.BlockSpec((2, H1), lambda i: (0, 0)),
      ],
      out_specs=[
          pl.BlockSpec((BLK, HP), lambda i: (i, 0)),
          pl.BlockSpec((BLK, 128), lambda i: (i % NBLK, 0)),
      ],
      out_shape=[
          jax.ShapeDtypeStruct((2 * NP, HP), jnp.float32),
          jax.ShapeDtypeStruct((NP, 128), jnp.float32),
      ],
  )(xp, w1, a1)


def _elu(v):
  return jnp.where(v > 0, v, jnp.exp(v) - 1.0)


def _tc_layer2(u1, w2, a2, b1r):
  def body(u_ref, w_ref, a_ref, b_ref, t_ref, sd_ref):
    i = pl.program_id(0)
    ua = u_ref[0]
    ub = u_ref[1]
    ha = ua[:, :128] / ua[:, 128:129]
    hb = ub[:, :128] / ub[:, 128:129]
    valid = _valid_col(i)
    x2 = jnp.concatenate([ha, hb], axis=1) + b_ref[...]
    x2 = jnp.where(valid > 0, _elu(x2), 0.0)
    h = jnp.dot(x2, w_ref[...], preferred_element_type=jnp.float32)
    sd_ref[...] = _sd_block(h, a_ref)
    t_ref[...] = jnp.concatenate(
        [h, valid, jnp.zeros((BLK, HP - 129), jnp.float32)], axis=1)

  return pl.pallas_call(
      body,
      grid=(NBLK,),
      in_specs=[
          pl.BlockSpec((2, BLK, HP), lambda i: (0, i, 0)),
          pl.BlockSpec((H1, H2), lambda i: (0, 0)),
          pl.BlockSpec((2, H2), lambda i: (0, 0)),
          pl.BlockSpec((1, H1), lambda i: (0, 0)),
      ],
      out_specs=[
          pl.BlockSpec((BLK, HP), lambda i: (i, 0)),
          pl.BlockSpec((BLK, 128), lambda i: (i, 0)),
      ],
      out_shape=[
          jax.ShapeDtypeStruct((NP, HP), jnp.float32),
          jax.ShapeDtypeStruct((NP, 128), jnp.float32),
      ],
  )(u1, w2, a2, b1r)


def _tc_layer3(u2, w3p, a3p, b2r):
  def body(u_ref, w_ref, a_ref, b_ref, t_ref, sd_ref):
    i = pl.program_id(0)
    acc = u_ref[0] + u_ref[1]
    valid = _valid_col(i)
    x3 = acc[:, :128] / acc[:, 128:129] + b_ref[...]
    x3 = jnp.where(valid > 0, _elu(x3), 0.0)
    h = jnp.dot(x3, w_ref[...], preferred_element_type=jnp.float32)
    s_col = jnp.sum(h * a_ref[0:1, :], axis=1, keepdims=True)
    d_col = jnp.sum(h * a_ref[1:2, :], axis=1, keepdims=True)
    col = lax.broadcasted_iota(jnp.int32, (BLK, 128), 1)
    sd_ref[...] = jnp.where(col == 0, s_col, jnp.where(col == 1, d_col, 0.0))
    col16 = lax.broadcasted_iota(jnp.int32, (BLK, HP3), 1)
    t_ref[...] = h + jnp.where(col16 == NCLS, valid, 0.0)

  return pl.pallas_call(
      body,
      grid=(NBLK,),
      in_specs=[
          pl.BlockSpec((2, BLK, HP), lambda i: (0, i, 0)),
          pl.BlockSpec((H2, HP3), lambda i: (0, 0)),
          pl.BlockSpec((2, HP3), lambda i: (0, 0)),
          pl.BlockSpec((1, H2), lambda i: (0, 0)),
      ],
      out_specs=[
          pl.BlockSpec((BLK, HP3), lambda i: (i, 0)),
          pl.BlockSpec((BLK, 128), lambda i: (i, 0)),
      ],
      out_shape=[
          jax.ShapeDtypeStruct((NP, HP3), jnp.float32),
          jax.ShapeDtypeStruct((NP, 128), jnp.float32),
      ],
  )(u2, w3p, a3p, b2r)


def _tc_final(u3, b3p):
  def body(u_ref, b_ref, o_ref):
    i = pl.program_id(0)
    acc = u_ref[0] + u_ref[1]
    valid = _valid_col(i)
    o = acc / acc[:, NCLS:NCLS + 1] + b_ref[...]
    o_ref[...] = jnp.where(valid > 0, o, 0.0)

  return pl.pallas_call(
      body,
      grid=(NBLK,),
      in_specs=[
          pl.BlockSpec((2, BLK, HP3), lambda i: (0, i, 0)),
          pl.BlockSpec((1, HP3), lambda i: (0, 0)),
      ],
      out_specs=pl.BlockSpec((BLK, HP3), lambda i: (i, 0)),
      out_shape=jax.ShapeDtypeStruct((NP, HP3), jnp.float32),
  )(u3, b3p)


def kernel(x, edge_index, W1, a1s, a1d, b1, W2, a2s, a2d, b2, W3, a3s, a3d,
           b3):
  loop = jnp.arange(N, dtype=jnp.int32)
  src = jnp.concatenate([edge_index[0].astype(jnp.int32), loop])
  dst = jnp.concatenate([edge_index[1].astype(jnp.int32), loop])
  pad = E2P - E2
  srcp = jnp.concatenate([src, jnp.full((pad,), N, jnp.int32)])
  dstp = jnp.concatenate([dst, jnp.full((pad,), N, jnp.int32)])
  src1 = jnp.concatenate([srcp, srcp + NP]).reshape(-1, EBATCH)
  dst1 = jnp.concatenate([dstp, dstp]).reshape(-1, EBATCH)
  src23 = srcp.reshape(-1, EBATCH)
  dst23 = dstp.reshape(-1, EBATCH)
  z144 = jnp.zeros((NP, HP), jnp.float32)
  z16 = jnp.zeros((NP, HP3), jnp.float32)

  xp = jnp.pad(x, ((0, NP - N), (0, 0)))
  a1 = jnp.stack([a1s, a1d])
  a2 = jnp.stack([a2s, a2d])
  w3p = jnp.pad(W3, ((0, 0), (0, HP3 - NCLS)))
  a3p = jnp.stack([jnp.pad(a3s, (0, HP3 - NCLS)),
                   jnp.pad(a3d, (0, HP3 - NCLS))])
  b1r = b1.reshape(1, H1)
  b2r = b2.reshape(1, H2)
  b3p = jnp.pad(b3, (0, HP3 - NCLS)).reshape(1, HP3)

  t1, sd1 = _tc_layer1(xp, W1, a1)
  u1 = _SC1(t1, sd1[:, 0], sd1[:, 1], src1, dst1, z144)

  t2, sd2 = _tc_layer2(u1.reshape(2, NP, HP), W2, a2, b1r)
  u2 = _SC2(t2, sd2[:, 0], sd2[:, 1], src23, dst23, z144)

  t3, sd3 = _tc_layer3(u2.reshape(2, NP, HP), w3p, a3p, b2r)
  u3 = _SC3(t3, sd3[:, 0], sd3[:, 1], srcp.reshape(-1, EBATCH3),
            dstp.reshape(-1, EBATCH3), z16)

  out = _tc_final(u3.reshape(2, NP, HP3), b3p)
  return out[:N, :NCLS]


# speedup vs baseline: 25.4003x; 1.0024x over previous
import functools

import jax
import jax.numpy as jnp
from jax import lax
from jax.experimental import pallas as pl
from jax.experimental.pallas import tpu as pltpu
from jax.experimental.pallas import tpu_sc as plsc

N = 10000
NP = 10240
E2 = 330000
EBATCH = 64
EBATCH3 = 128
E2P = 331776
D_IN, H1, H2, NCLS = 128, 256, 128, 7
HP = 144
HP3 = 16
BLK = 512
NBLK = NP // BLK
NPA = 10016
RPT = NPA // 16


def _sc_edge_sweep(hp, nb_core, nbt, src_off, eb, cb):
  g = hp // 16
  nchunk = nbt // cb
  mesh = plsc.VectorSubcoreMesh(core_axis_name="c", subcore_axis_name="s")

  @functools.partial(
      pl.kernel,
      out_type=jax.ShapeDtypeStruct((2 * NP, hp), jnp.float32),
      mesh=mesh,
      compiler_params=pltpu.CompilerParams(needs_layout_passes=False,
                                           use_tc_tiling_on_sc=False),
      scratch_types=[
          pltpu.VMEM((NP,), jnp.float32),
          pltpu.VMEM((NP,), jnp.float32),
          pltpu.VMEM((cb, eb), jnp.int32),
          pltpu.VMEM((cb, eb), jnp.int32),
          [pltpu.VMEM((eb, hp), jnp.float32)] * 2,
          [pltpu.VMEM((eb,), jnp.float32)] * 2,
          pltpu.VMEM_SHARED((NPA, hp), jnp.float32),
          [pltpu.SemaphoreType.DMA] * 2,
          [pltpu.SemaphoreType.DMA] * 2,
      ],
  )
  def sweep(table, s2, d2, srcs, dsts, zeros, out,
            s_v, d_v, src_ch, dst_ch, rows, ps, accum, gsem, ssem):
    c = lax.axis_index("c")
    t = lax.axis_index("s")
    off = c * src_off
    pltpu.sync_copy(s2, s_v)
    pltpu.sync_copy(d2, d_v)
    pltpu.sync_copy(zeros.at[pl.ds(t * RPT, RPT)],
                    accum.at[pl.ds(t * RPT, RPT)])
    plsc.subcore_barrier()

    def chunk(q, carry):
      row = c * nb_core + q * cb
      pltpu.sync_copy(srcs.at[pl.ds(row, cb)], src_ch)
      pltpu.sync_copy(dsts.at[pl.ds(row, cb)], dst_ch)
      gats = [None, None]
      scats = [None, None]
      gats[0] = pltpu.async_copy(table.at[src_ch.at[0]], rows[0], gsem[0])
      for b in range(cb):
        par = b % 2
        nxt = (b + 1) % 2
        if b + 1 < cb:
          if scats[nxt] is not None:
            scats[nxt].wait()
          gats[nxt] = pltpu.async_copy(table.at[src_ch.at[b + 1]],
                                       rows[nxt], gsem[nxt])
        for k in range(eb // 16):
          sl = pl.ds(k * 16, 16)
          e = (plsc.load_gather(s_v, [src_ch[b, sl] - off])
               + plsc.load_gather(d_v, [dst_ch[b, sl]]))
          e = jnp.where(e > 0, e, e * 0.2)
          ps[par][sl] = jnp.exp(e)
        gats[par].wait()

        def scale(i, carry2, par=par):
          pb = plsc.load_gather(ps[par], [jnp.full((16,), i, jnp.int32)])
          for v in range(g):
            sl2 = pl.ds(v * 16, 16)
            rows[par][i, sl2] = rows[par][i, sl2] * pb
          return carry2

        lax.fori_loop(0, eb, scale, 0, unroll=2)
        scats[par] = pltpu.async_copy(rows[par], accum.at[dst_ch.at[b]],
                                      ssem[par], add=True)
      scats[0].wait()
      scats[1].wait()
      return carry

    lax.fori_loop(t * nchunk, (t + 1) * nchunk, chunk, 0)
    plsc.subcore_barrier()
    pltpu.sync_copy(accum.at[pl.ds(t * RPT, RPT)],
                    out.at[pl.ds(c * NP + t * RPT, RPT)])

  return sweep


_SC1 = _sc_edge_sweep(HP, E2P // EBATCH, E2P // EBATCH // 16, NP, EBATCH, 6)
_SC2 = _sc_edge_sweep(HP, E2P // EBATCH // 2, E2P // EBATCH // 32, 0, EBATCH, 6)
_SC3 = _sc_edge_sweep(HP3, E2P // EBATCH3 // 2, E2P // EBATCH3 // 32, 0,
                      EBATCH3, 3)


def _valid_col(i):
  rows = i * BLK + lax.broadcasted_iota(jnp.int32, (BLK, 1), 0)
  return (rows < N).astype(jnp.float32)


def _sd_block(h, a_ref):
  s_col = jnp.sum(h * a_ref[0:1, :], axis=1, keepdims=True)
  d_col = jnp.sum(h * a_ref[1:2, :], axis=1, keepdims=True)
  col = lax.broadcasted_iota(jnp.int32, (BLK, 128), 1)
  return jnp.where(col == 0, s_col, jnp.where(col == 1, d_col, 0.0))


def _tc_layer1(xp, w1, a1):
  def body(x_ref, w_ref, a_ref, t_ref, sd_ref):
    i = pl.program_id(0)
    blk = i % NBLK
    h = jnp.dot(x_ref[...], w_ref[...], preferred_element_type=jnp.float32)
    valid = _valid_col(blk)
    sd_ref[...] = _sd_block(h, a_ref)
    hsel = jnp.where(i < NBLK, h[:, :128], h[:, 128:])
    t_ref[...] = jnp.concatenate(
        [hsel, valid, jnp.zeros((BLK, HP - 129), jnp.float32)], axis=1)

  return pl.pallas_call(
      body,
      grid=(2 * NBLK,),
      in_specs=[
          pl.BlockSpec((BLK, D_IN), lambda i: (i % NBLK, 0)),
          pl.BlockSpec((D_IN, H1), lambda i: (0, 0)),
          pl.BlockSpec((2, H1), lambda i: (0, 0)),
      ],
      out_specs=[
          pl.BlockSpec((BLK, HP), lambda i: (i, 0)),
          pl.BlockSpec((BLK, 128), lambda i: (i % NBLK, 0)),
      ],
      out_shape=[
          jax.ShapeDtypeStruct((2 * NP, HP), jnp.float32),
          jax.ShapeDtypeStruct((NP, 128), jnp.float32),
      ],
  )(xp, w1, a1)


def _elu(v):
  return jnp.where(v > 0, v, jnp.exp(v) - 1.0)


def _tc_layer2(u1, w2, a2, b1r):
  def body(u_ref, w_ref, a_ref, b_ref, t_ref, sd_ref):
    i = pl.program_id(0)
    ua = u_ref[0]
    ub = u_ref[1]
    ha = ua[:, :128] / ua[:, 128:129]
    hb = ub[:, :128] / ub[:, 128:129]
    valid = _valid_col(i)
    x2 = jnp.concatenate([ha, hb], axis=1) + b_ref[...]
    x2 = jnp.where(valid > 0, _elu(x2), 0.0)
    h = jnp.dot(x2, w_ref[...], preferred_element_type=jnp.float32)
    sd_ref[...] = _sd_block(h, a_ref)
    t_ref[...] = jnp.concatenate(
        [h, valid, jnp.zeros((BLK, HP - 129), jnp.float32)], axis=1)

  return pl.pallas_call(
      body,
      grid=(NBLK,),
      in_specs=[
          pl.BlockSpec((2, BLK, HP), lambda i: (0, i, 0)),
          pl.BlockSpec((H1, H2), lambda i: (0, 0)),
          pl.BlockSpec((2, H2), lambda i: (0, 0)),
          pl.BlockSpec((1, H1), lambda i: (0, 0)),
      ],
      out_specs=[
          pl.BlockSpec((BLK, HP), lambda i: (i, 0)),
          pl.BlockSpec((BLK, 128), lambda i: (i, 0)),
      ],
      out_shape=[
          jax.ShapeDtypeStruct((NP, HP), jnp.float32),
          jax.ShapeDtypeStruct((NP, 128), jnp.float32),
      ],
  )(u1, w2, a2, b1r)


def _tc_layer3(u2, w3p, a3p, b2r):
  def body(u_ref, w_ref, a_ref, b_ref, t_ref, sd_ref):
    i = pl.program_id(0)
    acc = u_ref[0] + u_ref[1]
    valid = _valid_col(i)
    x3 = acc[:, :128] / acc[:, 128:129] + b_ref[...]
    x3 = jnp.where(valid > 0, _elu(x3), 0.0)
    h = jnp.dot(x3, w_ref[...], preferred_element_type=jnp.float32)
    s_col = jnp.sum(h * a_ref[0:1, :], axis=1, keepdims=True)
    d_col = jnp.sum(h * a_ref[1:2, :], axis=1, keepdims=True)
    col = lax.broadcasted_iota(jnp.int32, (BLK, 128), 1)
    sd_ref[...] = jnp.where(col == 0, s_col, jnp.where(col == 1, d_col, 0.0))
    col16 = lax.broadcasted_iota(jnp.int32, (BLK, HP3), 1)
    t_ref[...] = h + jnp.where(col16 == NCLS, valid, 0.0)

  return pl.pallas_call(
      body,
      grid=(NBLK,),
      in_specs=[
          pl.BlockSpec((2, BLK, HP), lambda i: (0, i, 0)),
          pl.BlockSpec((H2, HP3), lambda i: (0, 0)),
          pl.BlockSpec((2, HP3), lambda i: (0, 0)),
          pl.BlockSpec((1, H2), lambda i: (0, 0)),
      ],
      out_specs=[
          pl.BlockSpec((BLK, HP3), lambda i: (i, 0)),
          pl.BlockSpec((BLK, 128), lambda i: (i, 0)),
      ],
      out_shape=[
          jax.ShapeDtypeStruct((NP, HP3), jnp.float32),
          jax.ShapeDtypeStruct((NP, 128), jnp.float32),
      ],
  )(u2, w3p, a3p, b2r)


def _tc_final(u3, b3p):
  def body(u_ref, b_ref, o_ref):
    i = pl.program_id(0)
    acc = u_ref[0] + u_ref[1]
    valid = _valid_col(i)
    o = acc / acc[:, NCLS:NCLS + 1] + b_ref[...]
    o_ref[...] = jnp.where(valid > 0, o, 0.0)

  return pl.pallas_call(
      body,
      grid=(NBLK,),
      in_specs=[
          pl.BlockSpec((2, BLK, HP3), lambda i: (0, i, 0)),
          pl.BlockSpec((1, HP3), lambda i: (0, 0)),
      ],
      out_specs=pl.BlockSpec((BLK, HP3), lambda i: (i, 0)),
      out_shape=jax.ShapeDtypeStruct((NP, HP3), jnp.float32),
  )(u3, b3p)


def kernel(x, edge_index, W1, a1s, a1d, b1, W2, a2s, a2d, b2, W3, a3s, a3d,
           b3):
  loop = jnp.arange(N, dtype=jnp.int32)
  src = jnp.concatenate([edge_index[0].astype(jnp.int32), loop])
  dst = jnp.concatenate([edge_index[1].astype(jnp.int32), loop])
  pad = E2P - E2
  srcp = jnp.concatenate([src, jnp.full((pad,), N, jnp.int32)])
  dstp = jnp.concatenate([dst, jnp.full((pad,), N, jnp.int32)])
  src1 = jnp.concatenate([srcp, srcp + NP]).reshape(-1, EBATCH)
  dst1 = jnp.concatenate([dstp, dstp]).reshape(-1, EBATCH)
  src23 = srcp.reshape(-1, EBATCH)
  dst23 = dstp.reshape(-1, EBATCH)
  z144 = jnp.zeros((NPA, HP), jnp.float32)
  z16 = jnp.zeros((NPA, HP3), jnp.float32)

  xp = jnp.pad(x, ((0, NP - N), (0, 0)))
  a1 = jnp.stack([a1s, a1d])
  a2 = jnp.stack([a2s, a2d])
  w3p = jnp.pad(W3, ((0, 0), (0, HP3 - NCLS)))
  a3p = jnp.stack([jnp.pad(a3s, (0, HP3 - NCLS)),
                   jnp.pad(a3d, (0, HP3 - NCLS))])
  b1r = b1.reshape(1, H1)
  b2r = b2.reshape(1, H2)
  b3p = jnp.pad(b3, (0, HP3 - NCLS)).reshape(1, HP3)

  t1, sd1 = _tc_layer1(xp, W1, a1)
  u1 = _SC1(t1, sd1[:, 0], sd1[:, 1], src1, dst1, z144)

  t2, sd2 = _tc_layer2(u1.reshape(2, NP, HP), W2, a2, b1r)
  u2 = _SC2(t2, sd2[:, 0], sd2[:, 1], src23, dst23, z144)

  t3, sd3 = _tc_layer3(u2.reshape(2, NP, HP), w3p, a3p, b2r)
  u3 = _SC3(t3, sd3[:, 0], sd3[:, 1], srcp.reshape(-1, EBATCH3),
            dstp.reshape(-1, EBATCH3), z16)

  out = _tc_final(u3.reshape(2, NP, HP3), b3p)
  return out[:N, :NCLS]


# speedup vs baseline: 27.9143x; 1.0990x over previous
import functools

import jax
import jax.numpy as jnp
from jax import lax
from jax.experimental import pallas as pl
from jax.experimental.pallas import tpu as pltpu
from jax.experimental.pallas import tpu_sc as plsc

N = 10000
NP = 10240
E2 = 330000
EBATCH = 64
EBATCH3 = 128
E2P = 331776
D_IN, H1, H2, NCLS = 128, 256, 128, 7
HP = 144
HP3 = 16
BLK = 512
NBLK = NP // BLK
NPA = 10016
RPT = NPA // 16


def _sc_edge_sweep(hp, nb_core, nbt, src_off, eb, cb):
  g = hp // 16
  nchunk = nbt // cb
  mesh = plsc.VectorSubcoreMesh(core_axis_name="c", subcore_axis_name="s")

  @functools.partial(
      pl.kernel,
      out_type=jax.ShapeDtypeStruct((2 * NP, hp), jnp.float32),
      mesh=mesh,
      compiler_params=pltpu.CompilerParams(needs_layout_passes=False,
                                           use_tc_tiling_on_sc=False),
      scratch_types=[
          pltpu.VMEM((NP,), jnp.float32),
          pltpu.VMEM((NP,), jnp.float32),
          pltpu.VMEM((cb, eb), jnp.int32),
          pltpu.VMEM((cb, eb), jnp.int32),
          [pltpu.VMEM((eb, hp), jnp.float32)] * 2,
          [pltpu.VMEM((eb,), jnp.float32)] * 2,
          pltpu.VMEM_SHARED((NPA, hp), jnp.float32),
          [pltpu.SemaphoreType.DMA] * 2,
          [pltpu.SemaphoreType.DMA] * 2,
      ],
  )
  def sweep(table, s2, d2, srcs, dsts, zeros, out,
            s_v, d_v, src_ch, dst_ch, rows, ps, accum, gsem, ssem):
    c = lax.axis_index("c")
    t = lax.axis_index("s")
    off = c * src_off
    pltpu.sync_copy(s2, s_v)
    pltpu.sync_copy(d2, d_v)
    pltpu.sync_copy(zeros.at[pl.ds(t * RPT, RPT)],
                    accum.at[pl.ds(t * RPT, RPT)])
    plsc.subcore_barrier()

    def chunk(q, carry):
      row = c * nb_core + q * cb
      pltpu.sync_copy(srcs.at[pl.ds(row, cb)], src_ch)
      pltpu.sync_copy(dsts.at[pl.ds(row, cb)], dst_ch)
      gats = [None, None]
      scats = [None, None]
      gats[0] = pltpu.async_copy(table.at[src_ch.at[0]], rows[0], gsem[0])
      for b in range(cb):
        par = b % 2
        nxt = (b + 1) % 2
        if b + 1 < cb:
          if scats[nxt] is not None:
            scats[nxt].wait()
          gats[nxt] = pltpu.async_copy(table.at[src_ch.at[b + 1]],
                                       rows[nxt], gsem[nxt])
        for k in range(eb // 16):
          sl = pl.ds(k * 16, 16)
          e = (plsc.load_gather(s_v, [src_ch[b, sl] - off])
               + plsc.load_gather(d_v, [dst_ch[b, sl]]))
          e = jnp.where(e > 0, e, e * 0.2)
          ps[par][sl] = jnp.exp(e)
        gats[par].wait()

        @plsc.parallel_loop(0, eb, unroll=4)
        def scale(i, par=par):
          pb = plsc.load_gather(ps[par], [jnp.full((16,), i, jnp.int32)])
          for v in range(g):
            sl2 = pl.ds(v * 16, 16)
            rows[par][i, sl2] = rows[par][i, sl2] * pb
        scats[par] = pltpu.async_copy(rows[par], accum.at[dst_ch.at[b]],
                                      ssem[par], add=True)
      scats[0].wait()
      scats[1].wait()
      return carry

    lax.fori_loop(t * nchunk, (t + 1) * nchunk, chunk, 0)
    plsc.subcore_barrier()
    pltpu.sync_copy(accum.at[pl.ds(t * RPT, RPT)],
                    out.at[pl.ds(c * NP + t * RPT, RPT)])

  return sweep


_SC1 = _sc_edge_sweep(HP, E2P // EBATCH, E2P // EBATCH // 16, NP, EBATCH, 6)
_SC2 = _sc_edge_sweep(HP, E2P // EBATCH // 2, E2P // EBATCH // 32, 0, EBATCH, 6)
_SC3 = _sc_edge_sweep(HP3, E2P // EBATCH3 // 2, E2P // EBATCH3 // 32, 0,
                      EBATCH3, 3)


def _valid_col(i):
  rows = i * BLK + lax.broadcasted_iota(jnp.int32, (BLK, 1), 0)
  return (rows < N).astype(jnp.float32)


def _sd_block(h, a_ref):
  s_col = jnp.sum(h * a_ref[0:1, :], axis=1, keepdims=True)
  d_col = jnp.sum(h * a_ref[1:2, :], axis=1, keepdims=True)
  col = lax.broadcasted_iota(jnp.int32, (BLK, 128), 1)
  return jnp.where(col == 0, s_col, jnp.where(col == 1, d_col, 0.0))


def _tc_layer1(xp, w1, a1):
  def body(x_ref, w_ref, a_ref, t_ref, sd_ref):
    i = pl.program_id(0)
    blk = i % NBLK
    h = jnp.dot(x_ref[...], w_ref[...], preferred_element_type=jnp.float32)
    valid = _valid_col(blk)
    sd_ref[...] = _sd_block(h, a_ref)
    hsel = jnp.where(i < NBLK, h[:, :128], h[:, 128:])
    t_ref[...] = jnp.concatenate(
        [hsel, valid, jnp.zeros((BLK, HP - 129), jnp.float32)], axis=1)

  return pl.pallas_call(
      body,
      grid=(2 * NBLK,),
      in_specs=[
          pl.BlockSpec((BLK, D_IN), lambda i: (i % NBLK, 0)),
          pl.BlockSpec((D_IN, H1), lambda i: (0, 0)),
          pl.BlockSpec((2, H1), lambda i: (0, 0)),
      ],
      out_specs=[
          pl.BlockSpec((BLK, HP), lambda i: (i, 0)),
          pl.BlockSpec((BLK, 128), lambda i: (i % NBLK, 0)),
      ],
      out_shape=[
          jax.ShapeDtypeStruct((2 * NP, HP), jnp.float32),
          jax.ShapeDtypeStruct((NP, 128), jnp.float32),
      ],
  )(xp, w1, a1)


def _elu(v):
  return jnp.where(v > 0, v, jnp.exp(v) - 1.0)


def _tc_layer2(u1, w2, a2, b1r):
  def body(u_ref, w_ref, a_ref, b_ref, t_ref, sd_ref):
    i = pl.program_id(0)
    ua = u_ref[0]
    ub = u_ref[1]
    ha = ua[:, :128] / ua[:, 128:129]
    hb = ub[:, :128] / ub[:, 128:129]
    valid = _valid_col(i)
    x2 = jnp.concatenate([ha, hb], axis=1) + b_ref[...]
    x2 = jnp.where(valid > 0, _elu(x2), 0.0)
    h = jnp.dot(x2, w_ref[...], preferred_element_type=jnp.float32)
    sd_ref[...] = _sd_block(h, a_ref)
    t_ref[...] = jnp.concatenate(
        [h, valid, jnp.zeros((BLK, HP - 129), jnp.float32)], axis=1)

  return pl.pallas_call(
      body,
      grid=(NBLK,),
      in_specs=[
          pl.BlockSpec((2, BLK, HP), lambda i: (0, i, 0)),
          pl.BlockSpec((H1, H2), lambda i: (0, 0)),
          pl.BlockSpec((2, H2), lambda i: (0, 0)),
          pl.BlockSpec((1, H1), lambda i: (0, 0)),
      ],
      out_specs=[
          pl.BlockSpec((BLK, HP), lambda i: (i, 0)),
          pl.BlockSpec((BLK, 128), lambda i: (i, 0)),
      ],
      out_shape=[
          jax.ShapeDtypeStruct((NP, HP), jnp.float32),
          jax.ShapeDtypeStruct((NP, 128), jnp.float32),
      ],
  )(u1, w2, a2, b1r)


def _tc_layer3(u2, w3p, a3p, b2r):
  def body(u_ref, w_ref, a_ref, b_ref, t_ref, sd_ref):
    i = pl.program_id(0)
    acc = u_ref[0] + u_ref[1]
    valid = _valid_col(i)
    x3 = acc[:, :128] / acc[:, 128:129] + b_ref[...]
    x3 = jnp.where(valid > 0, _elu(x3), 0.0)
    h = jnp.dot(x3, w_ref[...], preferred_element_type=jnp.float32)
    s_col = jnp.sum(h * a_ref[0:1, :], axis=1, keepdims=True)
    d_col = jnp.sum(h * a_ref[1:2, :], axis=1, keepdims=True)
    col = lax.broadcasted_iota(jnp.int32, (BLK, 128), 1)
    sd_ref[...] = jnp.where(col == 0, s_col, jnp.where(col == 1, d_col, 0.0))
    col16 = lax.broadcasted_iota(jnp.int32, (BLK, HP3), 1)
    t_ref[...] = h + jnp.where(col16 == NCLS, valid, 0.0)

  return pl.pallas_call(
      body,
      grid=(NBLK,),
      in_specs=[
          pl.BlockSpec((2, BLK, HP), lambda i: (0, i, 0)),
          pl.BlockSpec((H2, HP3), lambda i: (0, 0)),
          pl.BlockSpec((2, HP3), lambda i: (0, 0)),
          pl.BlockSpec((1, H2), lambda i: (0, 0)),
      ],
      out_specs=[
          pl.BlockSpec((BLK, HP3), lambda i: (i, 0)),
          pl.BlockSpec((BLK, 128), lambda i: (i, 0)),
      ],
      out_shape=[
          jax.ShapeDtypeStruct((NP, HP3), jnp.float32),
          jax.ShapeDtypeStruct((NP, 128), jnp.float32),
      ],
  )(u2, w3p, a3p, b2r)


def _tc_final(u3, b3p):
  def body(u_ref, b_ref, o_ref):
    i = pl.program_id(0)
    acc = u_ref[0] + u_ref[1]
    valid = _valid_col(i)
    o = acc / acc[:, NCLS:NCLS + 1] + b_ref[...]
    o_ref[...] = jnp.where(valid > 0, o, 0.0)

  return pl.pallas_call(
      body,
      grid=(NBLK,),
      in_specs=[
          pl.BlockSpec((2, BLK, HP3), lambda i: (0, i, 0)),
          pl.BlockSpec((1, HP3), lambda i: (0, 0)),
      ],
      out_specs=pl.BlockSpec((BLK, HP3), lambda i: (i, 0)),
      out_shape=jax.ShapeDtypeStruct((NP, HP3), jnp.float32),
  )(u3, b3p)


def kernel(x, edge_index, W1, a1s, a1d, b1, W2, a2s, a2d, b2, W3, a3s, a3d,
           b3):
  loop = jnp.arange(N, dtype=jnp.int32)
  src = jnp.concatenate([edge_index[0].astype(jnp.int32), loop])
  dst = jnp.concatenate([edge_index[1].astype(jnp.int32), loop])
  pad = E2P - E2
  srcp = jnp.concatenate([src, jnp.full((pad,), N, jnp.int32)])
  dstp = jnp.concatenate([dst, jnp.full((pad,), N, jnp.int32)])
  src1 = jnp.concatenate([srcp, srcp + NP]).reshape(-1, EBATCH)
  dst1 = jnp.concatenate([dstp, dstp]).reshape(-1, EBATCH)
  src23 = srcp.reshape(-1, EBATCH)
  dst23 = dstp.reshape(-1, EBATCH)
  z144 = jnp.zeros((NPA, HP), jnp.float32)
  z16 = jnp.zeros((NPA, HP3), jnp.float32)

  xp = jnp.pad(x, ((0, NP - N), (0, 0)))
  a1 = jnp.stack([a1s, a1d])
  a2 = jnp.stack([a2s, a2d])
  w3p = jnp.pad(W3, ((0, 0), (0, HP3 - NCLS)))
  a3p = jnp.stack([jnp.pad(a3s, (0, HP3 - NCLS)),
                   jnp.pad(a3d, (0, HP3 - NCLS))])
  b1r = b1.reshape(1, H1)
  b2r = b2.reshape(1, H2)
  b3p = jnp.pad(b3, (0, HP3 - NCLS)).reshape(1, HP3)

  t1, sd1 = _tc_layer1(xp, W1, a1)
  u1 = _SC1(t1, sd1[:, 0], sd1[:, 1], src1, dst1, z144)

  t2, sd2 = _tc_layer2(u1.reshape(2, NP, HP), W2, a2, b1r)
  u2 = _SC2(t2, sd2[:, 0], sd2[:, 1], src23, dst23, z144)

  t3, sd3 = _tc_layer3(u2.reshape(2, NP, HP), w3p, a3p, b2r)
  u3 = _SC3(t3, sd3[:, 0], sd3[:, 1], srcp.reshape(-1, EBATCH3),
            dstp.reshape(-1, EBATCH3), z16)

  out = _tc_final(u3.reshape(2, NP, HP3), b3p)
  return out[:N, :NCLS]
